# baseline (speedup 1.0000x reference)
import jax
import jax.numpy as jnp
from jax.experimental import pallas as pl
from jax.experimental.pallas import tpu as pltpu

B, T, D, H = 128, 2048, 512, 8
HD = D // H
BB = 2


def _sal_kernel(x_ref, mb_ref, w_ref, beff_ref, wg_ref, bg_ref, scale_ref,
                vec_ref, uexp_ref, uh_ref):
    lane = jax.lax.broadcasted_iota(jnp.int32, (H, D), 1)
    row = jax.lax.broadcasted_iota(jnp.int32, (H, D), 0)
    bd = (lane // HD == row).astype(jnp.float32)
    w = w_ref[...]
    x_all = x_ref[...].reshape(BB * T, D)
    s = jax.lax.dot_general(x_all, w, (((1,), (0,)), ((), ())),
                            preferred_element_type=jnp.float32)
    safe_all = s.T + beff_ref[...] + mb_ref[0]
    for b in range(BB):
        x = x_ref[b]
        safe = safe_all[:, b * T:(b + 1) * T]
        m = jnp.max(safe, axis=1, keepdims=True)
        p = jnp.exp(safe - m)
        l = jnp.sum(p, axis=1, keepdims=True)
        valid = (m > -1e8).astype(jnp.float32)
        pooled = jax.lax.dot_general(p, x, (((1,), (0,)), ((), ())),
                                     preferred_element_type=jnp.float32)
        vec_h = pooled / (l + 1e-6) * valid
        g = jnp.sum(vec_h * wg_ref[...], axis=1, keepdims=True) + bg_ref[0, 0]
        u = jax.nn.sigmoid(g) * valid
        vec = jnp.sum(vec_h * bd, axis=0, keepdims=True)
        ss = jnp.sum(vec * vec, axis=1, keepdims=True)
        inv = jax.lax.rsqrt(ss / D + 1e-6)
        vec_ref[0, b] = (vec * inv * scale_ref[...])[0]
        uexp_ref[0, b] = jnp.sum(u * bd, axis=0)
        uh_ref[0, b] = u[:, 0]


def kernel(x, mask_bool, temp, w_sal, b_sal, w_gate, b_gate, scale):
    temperature = jax.nn.softplus(temp) + 0.3
    w_eff = (w_sal / temperature[None, :]).astype(jnp.float32)
    b_eff = (b_sal / temperature).reshape(H, 1).astype(jnp.float32)
    maskbias = jnp.where(mask_bool, 0.0, -1e9).astype(jnp.float32)
    maskbias = maskbias.reshape(B // BB, 1, BB * T)
    wg_bd = (jnp.eye(H, dtype=jnp.float32)[:, :, None]
             * w_gate[:, 0][None, None, :]).reshape(H, D)
    bg = b_gate.reshape(1, 1).astype(jnp.float32)
    scale_row = scale.reshape(1, D).astype(jnp.float32)

    grid = (B // BB,)
    vec3, uexp3, uh3 = pl.pallas_call(
        _sal_kernel,
        grid=grid,
        in_specs=[
            pl.BlockSpec((BB, T, D), lambda i: (i, 0, 0)),
            pl.BlockSpec((1, 1, BB * T), lambda i: (i, 0, 0)),
            pl.BlockSpec((D, H), lambda i: (0, 0)),
            pl.BlockSpec((H, 1), lambda i: (0, 0)),
            pl.BlockSpec((H, D), lambda i: (0, 0)),
            pl.BlockSpec((1, 1), lambda i: (0, 0)),
            pl.BlockSpec((1, D), lambda i: (0, 0)),
        ],
        out_specs=[
            pl.BlockSpec((1, BB, D), lambda i: (i, 0, 0)),
            pl.BlockSpec((1, BB, D), lambda i: (i, 0, 0)),
            pl.BlockSpec((1, BB, H), lambda i: (i, 0, 0)),
        ],
        out_shape=[
            jax.ShapeDtypeStruct((B // BB, BB, D), jnp.float32),
            jax.ShapeDtypeStruct((B // BB, BB, D), jnp.float32),
            jax.ShapeDtypeStruct((B // BB, BB, H), jnp.float32),
        ],
        compiler_params=pltpu.CompilerParams(
            dimension_semantics=("arbitrary",),
            vmem_limit_bytes=48 * 1024 * 1024,
        ),
        name="salience_write_head",
    )(x, maskbias, w_eff, b_eff, wg_bd, bg, scale_row)

    return (vec3.reshape(B, D), uexp3.reshape(B, D), uh3.reshape(B, H))


# speedup vs baseline: 1.2468x; 1.2468x over previous
import jax
import jax.numpy as jnp
from jax.experimental import pallas as pl
from jax.experimental.pallas import tpu as pltpu

B, T, D, H = 128, 2048, 512, 8
HD = D // H
BB = 4


def _sal_kernel(x_ref, mb_ref, w_ref, beff_ref, wg_ref, bg_ref, scale_ref,
                vec_ref, uexp_ref, uh_ref):
    lane = jax.lax.broadcasted_iota(jnp.int32, (H, D), 1)
    row = jax.lax.broadcasted_iota(jnp.int32, (H, D), 0)
    bd = (lane // HD == row).astype(jnp.float32)
    w = w_ref[...]
    x_all = x_ref[...].reshape(BB * T, D)
    s_t = jax.lax.dot_general(w, x_all, (((0,), (1,)), ((), ())),
                              preferred_element_type=jnp.float32)
    safe_all = s_t + beff_ref[...] + mb_ref[0]
    for b in range(BB):
        x = x_ref[b]
        safe = safe_all[:, b * T:(b + 1) * T]
        m = jnp.max(safe, axis=1, keepdims=True)
        p = jnp.exp(safe - m)
        l = jnp.sum(p, axis=1, keepdims=True)
        valid = (m > -1e8).astype(jnp.float32)
        pooled = jax.lax.dot_general(p, x, (((1,), (0,)), ((), ())),
                                     preferred_element_type=jnp.float32)
        vec_h = pooled / (l + 1e-6) * valid
        g = jnp.sum(vec_h * wg_ref[...], axis=1, keepdims=True) + bg_ref[0, 0]
        u = jax.nn.sigmoid(g) * valid
        vec = jnp.sum(vec_h * bd, axis=0, keepdims=True)
        ss = jnp.sum(vec * vec, axis=1, keepdims=True)
        inv = jax.lax.rsqrt(ss / D + 1e-6)
        vec_ref[0, b] = (vec * inv * scale_ref[...])[0]
        uexp_ref[0, b] = jnp.sum(u * bd, axis=0)
        uh_ref[0, b] = u[:, 0]


def kernel(x, mask_bool, temp, w_sal, b_sal, w_gate, b_gate, scale):
    temperature = jax.nn.softplus(temp) + 0.3
    w_eff = (w_sal / temperature[None, :]).astype(jnp.float32)
    b_eff = (b_sal / temperature).reshape(H, 1).astype(jnp.float32)
    maskbias = jnp.where(mask_bool, 0.0, -1e9).astype(jnp.float32)
    maskbias = maskbias.reshape(B // BB, 1, BB * T)
    wg_bd = (jnp.eye(H, dtype=jnp.float32)[:, :, None]
             * w_gate[:, 0][None, None, :]).reshape(H, D)
    bg = b_gate.reshape(1, 1).astype(jnp.float32)
    scale_row = scale.reshape(1, D).astype(jnp.float32)

    grid = (B // BB,)
    vec3, uexp3, uh3 = pl.pallas_call(
        _sal_kernel,
        grid=grid,
        in_specs=[
            pl.BlockSpec((BB, T, D), lambda i: (i, 0, 0)),
            pl.BlockSpec((1, 1, BB * T), lambda i: (i, 0, 0)),
            pl.BlockSpec((D, H), lambda i: (0, 0)),
            pl.BlockSpec((H, 1), lambda i: (0, 0)),
            pl.BlockSpec((H, D), lambda i: (0, 0)),
            pl.BlockSpec((1, 1), lambda i: (0, 0)),
            pl.BlockSpec((1, D), lambda i: (0, 0)),
        ],
        out_specs=[
            pl.BlockSpec((1, BB, D), lambda i: (i, 0, 0)),
            pl.BlockSpec((1, BB, D), lambda i: (i, 0, 0)),
            pl.BlockSpec((1, BB, H), lambda i: (i, 0, 0)),
        ],
        out_shape=[
            jax.ShapeDtypeStruct((B // BB, BB, D), jnp.float32),
            jax.ShapeDtypeStruct((B // BB, BB, D), jnp.float32),
            jax.ShapeDtypeStruct((B // BB, BB, H), jnp.float32),
        ],
        compiler_params=pltpu.CompilerParams(
            dimension_semantics=("arbitrary",),
            vmem_limit_bytes=48 * 1024 * 1024,
        ),
        name="salience_write_head",
    )(x, maskbias, w_eff, b_eff, wg_bd, bg, scale_row)

    return (vec3.reshape(B, D), uexp3.reshape(B, D), uh3.reshape(B, H))
